# baseline (speedup 1.0000x reference)
import functools

import jax
import jax.numpy as jnp
from jax import lax
from jax.experimental import pallas as pl
from jax.experimental.pallas import tpu as pltpu
from jax.experimental.pallas import tpu_sc as plsc

_BATCH = 16384
_HIST = 50
_NDIM = 32
_NSTIM = 1000000

_NC = 2
_NS = 16
_NW = _NC * _NS

_KCOL = 512
_NGRP = (_NSTIM // 128) * 128 // _KCOL
_TAIL0 = _NGRP * _KCOL
_TAILN = _NSTIM - _TAIL0
_K1_PER_W = -(-_NGRP // _NW)
_K1_PAIRS = (_K1_PER_W + 1) // 2
_PITCH = _NDIM + 1

_BPW = _BATCH // _NW
_G = 128
_NG = _BPW // _G
_OPITCH = _BPW + 9


def _build_k1():
    mesh = plsc.VectorSubcoreMesh(core_axis_name="c", subcore_axis_name="s")

    @functools.partial(
        pl.kernel,
        mesh=mesh,
        out_type=jax.ShapeDtypeStruct((_NSTIM * _NDIM,), jnp.float32),
        compiler_params=pltpu.CompilerParams(needs_layout_passes=False),
        scratch_types=[
            pltpu.VMEM((4, 8, _KCOL), jnp.float32),
            pltpu.VMEM((4, 8, _KCOL), jnp.float32),
            pltpu.VMEM((_KCOL * _PITCH,), jnp.float32),
            pltpu.VMEM((_KCOL * _NDIM,), jnp.float32),
            pltpu.VMEM((_KCOL * _NDIM,), jnp.float32),
            pltpu.VMEM((_TAILN * _NDIM,), jnp.float32),
            pltpu.SemaphoreType.DMA,
            pltpu.SemaphoreType.DMA,
            pltpu.SemaphoreType.DMA,
            pltpu.SemaphoreType.DMA,
        ],
    )
    def k1(zt_hbm, ztail_hbm, zlin_hbm, va, vb, tpad, ta, tb, tailv,
           sga, sgb, soa, sob):
        wid = lax.axis_index("s") * _NC + lax.axis_index("c")

        iota_p = jnp.arange(16, dtype=jnp.int32) * _PITCH

        def start_read(k, v, sg):
            c0 = k * _KCOL
            for j in range(4):
                pltpu.async_copy(
                    zt_hbm.at[pl.ds(j * 8, 8), pl.ds(c0, _KCOL)], v.at[j], sg
                )

        def wait_read(v, sg):
            for j in range(4):
                pltpu.make_async_copy(
                    zt_hbm.at[pl.ds(0, 8), pl.ds(0, _KCOL)], v.at[j], sg
                ).wait()

        def transpose_group(v, t):
            def scat(i, carry):
                j = i >> 3
                s = i & 7
                d = i
                for cc in range(_KCOL // 16):
                    vec = v[j, s, pl.ds(cc * 16, 16)]
                    plsc.store_scatter(
                        tpad, [iota_p + (cc * 16 * _PITCH + d)], vec
                    )
                return carry

            lax.fori_loop(0, _NDIM, scat, 0)

            def compact(i, carry):
                for cc in range(16):
                    c = i * 16 + cc
                    t[pl.ds(c * _NDIM, 16)] = tpad[pl.ds(c * _PITCH, 16)]
                    t[pl.ds(c * _NDIM + 16, 16)] = tpad[
                        pl.ds(c * _PITCH + 16, 16)
                    ]
                return carry

            lax.fori_loop(0, _KCOL // 16, compact, 0)

        def start_write(k, t, so):
            pltpu.async_copy(
                t, zlin_hbm.at[pl.ds(k * (_KCOL * _NDIM), _KCOL * _NDIM)], so
            )

        def wait_write(t, so):
            pltpu.make_async_copy(
                zlin_hbm.at[pl.ds(0, _KCOL * _NDIM)], t, so
            ).wait()

        def chunk_id(p, half):
            return wid + _NW * (2 * p + half)

        @pl.when(chunk_id(0, 0) < _NGRP)
        def _():
            start_read(chunk_id(0, 0), va, sga)

        def pair(p, carry):
            ka = chunk_id(p, 0)
            kb = chunk_id(p, 1)

            @pl.when(kb < _NGRP)
            def _():
                start_read(kb, vb, sgb)

            @pl.when(ka < _NGRP)
            def _():
                wait_read(va, sga)

                @pl.when(p > 0)
                def _():
                    wait_write(ta, soa)

                transpose_group(va, ta)
                start_write(ka, ta, soa)

            @pl.when(chunk_id(p + 1, 0) < _NGRP)
            def _():
                start_read(chunk_id(p + 1, 0), va, sga)

            @pl.when(kb < _NGRP)
            def _():
                wait_read(vb, sgb)

                @pl.when(p > 0)
                def _():
                    wait_write(tb, sob)

                transpose_group(vb, tb)
                start_write(kb, tb, sob)

            return carry

        lax.fori_loop(0, _K1_PAIRS, pair, 0)
        wait_write(ta, soa)
        wait_write(tb, sob)

        @pl.when(wid == 0)
        def _():
            pltpu.sync_copy(ztail_hbm, tailv)
            pltpu.sync_copy(
                tailv, zlin_hbm.at[pl.ds(_TAIL0 * _NDIM, _TAILN * _NDIM)]
            )

    return k1


def _build_k2():
    mesh = plsc.VectorSubcoreMesh(core_axis_name="c", subcore_axis_name="s")

    @functools.partial(
        pl.kernel,
        mesh=mesh,
        out_type=jax.ShapeDtypeStruct((_HIST, _NDIM, _BATCH), jnp.float32),
        compiler_params=pltpu.CompilerParams(
            needs_layout_passes=False, use_tc_tiling_on_sc=False
        ),
        scratch_types=[
            pltpu.VMEM((_BPW,), jnp.int32),
            pltpu.VMEM((_BPW,), jnp.int32),
            pltpu.VMEM((_BPW, _NDIM), jnp.float32),
            pltpu.VMEM((_BPW, _NDIM), jnp.float32),
            pltpu.VMEM((_NDIM, _OPITCH), jnp.float32),
            pltpu.VMEM((_NDIM, _OPITCH), jnp.float32),
            pltpu.SemaphoreType.DMA,
            pltpu.SemaphoreType.DMA,
            pltpu.SemaphoreType.DMA,
            pltpu.SemaphoreType.DMA,
        ],
    )
    def k2(idxt_hbm, z_hbm, out_hbm, idx0, idx1, rows0, rows1,
           ob0, ob1, sg0, sg1, so0, so1):
        wid = lax.axis_index("s") * _NC + lax.axis_index("c")
        b0w = wid * _BPW

        iota_dlo = jnp.arange(16, dtype=jnp.int32)
        iota_dhi = iota_dlo + 16

        def start(h, idx_v, rows_v, sg):
            pltpu.sync_copy(idxt_hbm.at[h, pl.ds(b0w, _BPW)], idx_v)
            for j in range(_NG):
                pltpu.async_copy(
                    z_hbm.at[idx_v.at[pl.ds(j * _G, _G)]],
                    rows_v.at[pl.ds(j * _G, _G)],
                    sg,
                )

        def wait_gather(rows_v, sg):
            pltpu.make_async_copy(
                z_hbm.at[pl.ds(0, _BPW)], rows_v, sg
            ).wait()

        def relayout(rows_v, out_v):
            def bbody(i, carry):
                for bb in range(8):
                    b = i * 8 + bb
                    b_splat = jnp.full((16,), 0, jnp.int32) + b
                    lo = rows_v[b, pl.ds(0, 16)]
                    hi = rows_v[b, pl.ds(16, 16)]
                    plsc.store_scatter(out_v, [iota_dlo, b_splat], lo)
                    plsc.store_scatter(out_v, [iota_dhi, b_splat], hi)
                return carry

            lax.fori_loop(0, _BPW // 8, bbody, 0)

        def start_out(h, out_v, so):
            pltpu.async_copy(
                out_v.at[:, pl.ds(0, _BPW)],
                out_hbm.at[h, :, pl.ds(b0w, _BPW)],
                so,
            )

        def wait_out(out_v, so):
            pltpu.make_async_copy(
                out_hbm.at[0, :, pl.ds(0, _BPW)],
                out_v.at[:, pl.ds(0, _BPW)],
                so,
            ).wait()

        start(0, idx0, rows0, sg0)

        def pair(p, carry):
            h0 = 2 * p
            h1 = h0 + 1
            start(h1, idx1, rows1, sg1)
            wait_gather(rows0, sg0)

            @pl.when(p > 0)
            def _():
                wait_out(ob0, so0)

            relayout(rows0, ob0)
            start_out(h0, ob0, so0)

            @pl.when(p < _HIST // 2 - 1)
            def _():
                start(h0 + 2, idx0, rows0, sg0)

            wait_gather(rows1, sg1)

            @pl.when(p > 0)
            def _():
                wait_out(ob1, so1)

            relayout(rows1, ob1)
            start_out(h1, ob1, so1)
            return carry

        lax.fori_loop(0, _HIST // 2, pair, 0)
        wait_out(ob0, so0)
        wait_out(ob1, so1)

    return k2


_K1 = _build_k1()
_K2 = _build_k2()


@jax.jit
def kernel(inputs, z):
    zt = jnp.transpose(z)
    ztail = lax.slice(z, (_TAIL0, 0), (_NSTIM, _NDIM)).reshape(-1)
    z_lin = _K1(zt, ztail).reshape(_NSTIM, _NDIM)
    idxt = jnp.transpose(inputs)
    out_t = _K2(idxt, z_lin)
    return jnp.transpose(out_t, (2, 1, 0))


# speedup vs baseline: 1.0343x; 1.0343x over previous
import functools

import jax
import jax.numpy as jnp
from jax import lax
from jax.experimental import pallas as pl
from jax.experimental.pallas import tpu as pltpu
from jax.experimental.pallas import tpu_sc as plsc

_BATCH = 16384
_HIST = 50
_NDIM = 32
_NSTIM = 1000000

_NC = 2
_NS = 16
_NW = _NC * _NS

_KCOL = 512
_NGRP = (_NSTIM // 128) * 128 // _KCOL
_TAIL0 = _NGRP * _KCOL
_TAILN = _NSTIM - _TAIL0
_K1_PER_W = -(-_NGRP // _NW)
_K1_PAIRS = (_K1_PER_W + 1) // 2
_PITCH = _NDIM + 1

_BPW = _BATCH // _NW
_G = 128
_NG = _BPW // _G
_OPITCH = _BPW + 9


def _build_k1():
    mesh = plsc.VectorSubcoreMesh(core_axis_name="c", subcore_axis_name="s")

    @functools.partial(
        pl.kernel,
        mesh=mesh,
        out_type=jax.ShapeDtypeStruct((_NSTIM * _NDIM,), jnp.float32),
        compiler_params=pltpu.CompilerParams(needs_layout_passes=False),
        scratch_types=[
            pltpu.VMEM((4, 8, _KCOL), jnp.float32),
            pltpu.VMEM((4, 8, _KCOL), jnp.float32),
            pltpu.VMEM((_KCOL * _PITCH,), jnp.float32),
            pltpu.VMEM((_KCOL * _NDIM,), jnp.float32),
            pltpu.VMEM((_KCOL * _NDIM,), jnp.float32),
            pltpu.VMEM((_TAILN * _NDIM,), jnp.float32),
            pltpu.SemaphoreType.DMA,
            pltpu.SemaphoreType.DMA,
            pltpu.SemaphoreType.DMA,
            pltpu.SemaphoreType.DMA,
        ],
    )
    def k1(zt_hbm, ztail_hbm, zlin_hbm, va, vb, tpad, ta, tb, tailv,
           sga, sgb, soa, sob):
        wid = lax.axis_index("s") * _NC + lax.axis_index("c")

        iota_p = jnp.arange(16, dtype=jnp.int32) * _PITCH

        def start_read(k, v, sg):
            c0 = k * _KCOL
            for j in range(4):
                pltpu.async_copy(
                    zt_hbm.at[pl.ds(j * 8, 8), pl.ds(c0, _KCOL)], v.at[j], sg
                )

        def wait_read(v, sg):
            for j in range(4):
                pltpu.make_async_copy(
                    zt_hbm.at[pl.ds(0, 8), pl.ds(0, _KCOL)], v.at[j], sg
                ).wait()

        def transpose_group(v, t):
            def scat(cc, carry):
                base = cc * (16 * _PITCH)
                off = cc * 16
                for j in range(4):
                    for s in range(8):
                        vec = v[j, s, pl.ds(off, 16)]
                        plsc.store_scatter(
                            tpad, [iota_p + (base + (j * 8 + s))], vec
                        )
                return carry

            lax.fori_loop(0, _KCOL // 16, scat, 0)

            def compact(i, carry):
                for cc in range(16):
                    c = i * 16 + cc
                    t[pl.ds(c * _NDIM, 16)] = tpad[pl.ds(c * _PITCH, 16)]
                    t[pl.ds(c * _NDIM + 16, 16)] = tpad[
                        pl.ds(c * _PITCH + 16, 16)
                    ]
                return carry

            lax.fori_loop(0, _KCOL // 16, compact, 0)

        def start_write(k, t, so):
            pltpu.async_copy(
                t, zlin_hbm.at[pl.ds(k * (_KCOL * _NDIM), _KCOL * _NDIM)], so
            )

        def wait_write(t, so):
            pltpu.make_async_copy(
                zlin_hbm.at[pl.ds(0, _KCOL * _NDIM)], t, so
            ).wait()

        def chunk_id(p, half):
            return wid + _NW * (2 * p + half)

        @pl.when(chunk_id(0, 0) < _NGRP)
        def _():
            start_read(chunk_id(0, 0), va, sga)

        def pair(p, carry):
            ka = chunk_id(p, 0)
            kb = chunk_id(p, 1)

            @pl.when(kb < _NGRP)
            def _():
                start_read(kb, vb, sgb)

            @pl.when(ka < _NGRP)
            def _():
                wait_read(va, sga)

                @pl.when(p > 0)
                def _():
                    wait_write(ta, soa)

                transpose_group(va, ta)
                start_write(ka, ta, soa)

            @pl.when(chunk_id(p + 1, 0) < _NGRP)
            def _():
                start_read(chunk_id(p + 1, 0), va, sga)

            @pl.when(kb < _NGRP)
            def _():
                wait_read(vb, sgb)

                @pl.when(p > 0)
                def _():
                    wait_write(tb, sob)

                transpose_group(vb, tb)
                start_write(kb, tb, sob)

            return carry

        lax.fori_loop(0, _K1_PAIRS, pair, 0)
        wait_write(ta, soa)
        wait_write(tb, sob)

        @pl.when(wid == 0)
        def _():
            pltpu.sync_copy(ztail_hbm, tailv)
            pltpu.sync_copy(
                tailv, zlin_hbm.at[pl.ds(_TAIL0 * _NDIM, _TAILN * _NDIM)]
            )

    return k1


def _build_k2():
    mesh = plsc.VectorSubcoreMesh(core_axis_name="c", subcore_axis_name="s")

    @functools.partial(
        pl.kernel,
        mesh=mesh,
        out_type=jax.ShapeDtypeStruct((_HIST, _NDIM, _BATCH), jnp.float32),
        compiler_params=pltpu.CompilerParams(
            needs_layout_passes=False, use_tc_tiling_on_sc=False
        ),
        scratch_types=[
            pltpu.VMEM((_HIST, _BPW), jnp.int32),
            pltpu.VMEM((_BPW, _NDIM), jnp.float32),
            pltpu.VMEM((_BPW, _NDIM), jnp.float32),
            pltpu.VMEM((_NDIM, _OPITCH), jnp.float32),
            pltpu.VMEM((_NDIM, _OPITCH), jnp.float32),
            pltpu.SemaphoreType.DMA,
            pltpu.SemaphoreType.DMA,
            pltpu.SemaphoreType.DMA,
            pltpu.SemaphoreType.DMA,
        ],
    )
    def k2(idxt_hbm, z_hbm, out_hbm, idx_all, rows0, rows1,
           ob0, ob1, sg0, sg1, so0, so1):
        wid = lax.axis_index("s") * _NC + lax.axis_index("c")
        b0w = wid * _BPW

        iota_dlo = jnp.arange(16, dtype=jnp.int32)
        iota_dhi = iota_dlo + 16

        pltpu.sync_copy(idxt_hbm.at[:, pl.ds(b0w, _BPW)], idx_all)

        def start(h, rows_v, sg):
            for j in range(_NG):
                pltpu.async_copy(
                    z_hbm.at[idx_all.at[h, pl.ds(j * _G, _G)]],
                    rows_v.at[pl.ds(j * _G, _G)],
                    sg,
                )

        def wait_gather(rows_v, sg):
            pltpu.make_async_copy(
                z_hbm.at[pl.ds(0, _BPW)], rows_v, sg
            ).wait()

        def relayout(rows_v, out_v):
            def bbody(i, carry):
                for bb in range(8):
                    b = i * 8 + bb
                    b_splat = jnp.full((16,), 0, jnp.int32) + b
                    lo = rows_v[b, pl.ds(0, 16)]
                    hi = rows_v[b, pl.ds(16, 16)]
                    plsc.store_scatter(out_v, [iota_dlo, b_splat], lo)
                    plsc.store_scatter(out_v, [iota_dhi, b_splat], hi)
                return carry

            lax.fori_loop(0, _BPW // 8, bbody, 0)

        def start_out(h, out_v, so):
            pltpu.async_copy(
                out_v.at[:, pl.ds(0, _BPW)],
                out_hbm.at[h, :, pl.ds(b0w, _BPW)],
                so,
            )

        def wait_out(out_v, so):
            pltpu.make_async_copy(
                out_hbm.at[0, :, pl.ds(0, _BPW)],
                out_v.at[:, pl.ds(0, _BPW)],
                so,
            ).wait()

        start(0, rows0, sg0)

        def pair(p, carry):
            h0 = 2 * p
            h1 = h0 + 1
            start(h1, rows1, sg1)
            wait_gather(rows0, sg0)

            @pl.when(p > 0)
            def _():
                wait_out(ob0, so0)

            relayout(rows0, ob0)
            start_out(h0, ob0, so0)

            @pl.when(p < _HIST // 2 - 1)
            def _():
                start(h0 + 2, rows0, sg0)

            wait_gather(rows1, sg1)

            @pl.when(p > 0)
            def _():
                wait_out(ob1, so1)

            relayout(rows1, ob1)
            start_out(h1, ob1, so1)
            return carry

        lax.fori_loop(0, _HIST // 2, pair, 0)
        wait_out(ob0, so0)
        wait_out(ob1, so1)

    return k2


_K1 = _build_k1()
_K2 = _build_k2()


@jax.jit
def kernel(inputs, z):
    zt = jnp.transpose(z)
    ztail = lax.slice(z, (_TAIL0, 0), (_NSTIM, _NDIM)).reshape(-1)
    z_lin = _K1(zt, ztail).reshape(_NSTIM, _NDIM)
    idxt = jnp.transpose(inputs)
    out_t = _K2(idxt, z_lin)
    return jnp.transpose(out_t, (2, 1, 0))
